# baseline (speedup 1.0000x reference)
import dataclasses
import functools

import jax
import jax.numpy as jnp
from jax import lax
from jax.experimental import pallas as pl
from jax.experimental.pallas import tpu as pltpu
from jax.experimental.pallas import tpu_sc as plsc

N_NODES = 10000
N_EDGES = 320000
D_FEAT = 128

NUM_CORES = 2
NUM_SUBCORES = 16
NUM_WORKERS = NUM_CORES * NUM_SUBCORES
EPW = N_EDGES // NUM_WORKERS
K = 48
NCHUNK = EPW // K
TAIL = EPW - NCHUNK * K
NBUF = 4
ZROWS = 624
ZREM = N_NODES - NUM_SUBCORES * ZROWS
L = 16


def _score_body(w2_ref, h_ref, b2_ref, d_ref, out_ref, hd_ref):
    out_ref[...] = lax.dot_general(
        w2_ref[...], h_ref[...], (((1,), (1,)), ((), ())),
        preferred_element_type=jnp.float32,
        precision=lax.Precision.HIGHEST,
    ) + b2_ref[...]
    hd_ref[...] = h_ref[...] * d_ref[...]


def _add_body(zp_ref, d_ref, out_ref):
    out_ref[...] = (zp_ref[0] + zp_ref[1]) * d_ref[...]


def _sc_body(src_hbm, dst_hbm, a_hbm, s_hbm, hd_hbm, out_hbm,
             a_v, s_v, rows0, rows1, rows2, rows3,
             srcc0, srcc1, srcc2, srcc3, dstc0, dstc1, dstc2, dstc3,
             sdst0, sdst1, sdst2, sdst3, tdst_v, z_sh,
             sg0, sg1, sg2, sg3, ss0, ss1, ss2, ss3,
             sis0, sis1, sis2, sis3, sid0, sid1, sid2, sid3):
    rows = (rows0, rows1, rows2, rows3)
    srcc = (srcc0, srcc1, srcc2, srcc3)
    dstc = (dstc0, dstc1, dstc2, dstc3)
    sdst = (sdst0, sdst1, sdst2, sdst3)
    sg = (sg0, sg1, sg2, sg3)
    ss = (ss0, ss1, ss2, ss3)
    sis = (sis0, sis1, sis2, sis3)
    sid_ = (sid0, sid1, sid2, sid3)

    cid = lax.axis_index("c")
    sid = lax.axis_index("s")
    w = cid * NUM_SUBCORES + sid
    ebase = w * EPW

    def idx_start(b, cc):
        eb = ebase + cc * K
        pltpu.make_async_copy(src_hbm.at[pl.ds(eb, K)], srcc[b], sis[b]).start()
        pltpu.make_async_copy(dst_hbm.at[pl.ds(eb, K)], dstc[b],
                              sid_[b]).start()

    def idx_wait(b):
        pltpu.make_async_copy(src_hbm.at[pl.ds(0, K)], srcc[b], sis[b]).wait()
        pltpu.make_async_copy(dst_hbm.at[pl.ds(0, K)], dstc[b],
                              sid_[b]).wait()

    def gather_start(b):
        pltpu.make_async_copy(hd_hbm.at[srcc[b]], rows[b], sg[b]).start()

    def gather_wait(b):
        pltpu.make_async_copy(hd_hbm.at[srcc[b]], rows[b], sg[b]).wait()

    def scat_start(b):
        pltpu.make_async_copy(rows[b], z_sh.at[sdst[b]], ss[b]).start(add=True)

    def scat_wait(b):
        pltpu.make_async_copy(rows[b], z_sh.at[sdst[b]], ss[b]).wait()

    def compute(b):
        srcc_b, dstc_b, sdst_b, rows_b = srcc[b], dstc[b], sdst[b], rows[b]

        @pl.loop(0, K, step=L)
        def _scale(i0):
            srcv = srcc_b[pl.ds(i0, L)]
            dstv = dstc_b[pl.ds(i0, L)]
            sdst_b[pl.ds(i0, L)] = dstv
            x = plsc.load_gather(a_v, [dstv]) + plsc.load_gather(s_v, [srcv])
            t = jnp.exp(-2.0 * jnp.abs(x))
            e16 = jnp.sign(x) * (1.0 - t) / (1.0 + t)
            for lane in range(L):
                ev = jnp.full((L,), e16[lane], jnp.float32)
                i = i0 + lane
                for j in range(D_FEAT // L):
                    rows_b[i, pl.ds(j * L, L)] = rows_b[i, pl.ds(j * L, L)] * ev

    idx_start(0, 0)
    idx_start(1, 1)
    idx_start(2, 2)

    pltpu.sync_copy(a_hbm, a_v)
    pltpu.sync_copy(s_hbm, s_v)

    @pl.loop(0, K)
    def _zero_rows(i):
        for j in range(D_FEAT // L):
            rows0[i, pl.ds(j * L, L)] = jnp.zeros((L,), jnp.float32)

    for t in range(ZROWS // K):
        pltpu.sync_copy(rows0, z_sh.at[pl.ds(sid * ZROWS + t * K, K)])
    ztail = ZROWS - (ZROWS // K) * K
    if ztail:
        pltpu.sync_copy(rows0.at[pl.ds(0, ztail)],
                        z_sh.at[pl.ds(sid * ZROWS + ZROWS - ztail, ztail)])

    @pl.when(sid == NUM_SUBCORES - 1)
    def _zero_rem():
        pltpu.sync_copy(rows0.at[pl.ds(0, ZREM)],
                        z_sh.at[pl.ds(NUM_SUBCORES * ZROWS, ZREM)])

    idx_wait(0)
    gather_start(0)
    idx_wait(1)
    gather_start(1)

    plsc.subcore_barrier()

    @pl.loop(0, NCHUNK, step=NBUF)
    def _msg(c):
        for p in range(NBUF):
            cc = c + p
            b = p

            gather_wait(b)

            @pl.when(cc >= 1)
            def _w_scat():
                scat_wait((b + 3) % NBUF)

            @pl.when(cc + 3 < NCHUNK)
            def _i_next():
                idx_start((b + 3) % NBUF, cc + 3)

            @pl.when(cc + 2 < NCHUNK)
            def _g_next():
                idx_wait((b + 2) % NBUF)
                gather_start((b + 2) % NBUF)

            compute(b)
            scat_start(b)

    tb = ebase + NCHUNK * K
    pltpu.sync_copy(src_hbm.at[pl.ds(tb, TAIL)], srcc0.at[pl.ds(0, TAIL)])
    pltpu.sync_copy(dst_hbm.at[pl.ds(tb, TAIL)], tdst_v)
    pltpu.sync_copy(hd_hbm.at[srcc0.at[pl.ds(0, TAIL)]],
                    rows0.at[pl.ds(0, TAIL)])
    srcv = srcc0[pl.ds(0, L)]
    dstv = tdst_v[pl.ds(0, L)]
    x = plsc.load_gather(a_v, [dstv]) + plsc.load_gather(s_v, [srcv])
    t = jnp.exp(-2.0 * jnp.abs(x))
    e16 = jnp.sign(x) * (1.0 - t) / (1.0 + t)
    for lane in range(L):
        ev = jnp.full((L,), e16[lane], jnp.float32)
        for j in range(D_FEAT // L):
            rows0[lane, pl.ds(j * L, L)] = rows0[lane, pl.ds(j * L, L)] * ev
    pltpu.sync_copy(rows0.at[pl.ds(0, TAIL)], z_sh.at[tdst_v], add=True)
    scat_wait(NBUF - 1)

    plsc.subcore_barrier()

    pltpu.sync_copy(z_sh.at[pl.ds(sid * ZROWS, ZROWS)],
                    out_hbm.at[cid, pl.ds(sid * ZROWS, ZROWS)])

    @pl.when(sid == NUM_SUBCORES - 1)
    def _copy_rem():
        pltpu.sync_copy(z_sh.at[pl.ds(NUM_SUBCORES * ZROWS, ZREM)],
                        out_hbm.at[cid, pl.ds(NUM_SUBCORES * ZROWS, ZREM)])


def kernel(h, edge_index, d, gate_w, gate_b):
    src = edge_index[0].astype(jnp.int32)
    dst = edge_index[1].astype(jnp.int32)

    w2 = gate_w.reshape(2, D_FEAT)
    b2 = jnp.concatenate([gate_b, jnp.zeros((1,), jnp.float32)])[:, None]

    scores, hd = pl.pallas_call(
        _score_body,
        out_shape=(jax.ShapeDtypeStruct((2, N_NODES), jnp.float32),
                   jax.ShapeDtypeStruct((N_NODES, D_FEAT), jnp.float32)),
    )(w2, h, b2, d[:, None])

    mesh = plsc.VectorSubcoreMesh(core_axis_name="c", subcore_axis_name="s")
    cp = pltpu.CompilerParams()
    if "needs_layout_passes" in pltpu.CompilerParams.__dataclass_fields__:
        cp = dataclasses.replace(cp, needs_layout_passes=False)
    sc_kernel = functools.partial(
        pl.kernel,
        compiler_params=cp,
        out_type=jax.ShapeDtypeStruct((NUM_CORES, N_NODES, D_FEAT),
                                      jnp.float32),
        mesh=mesh,
        scratch_types=[
            pltpu.VMEM((N_NODES,), jnp.float32),
            pltpu.VMEM((N_NODES,), jnp.float32),
        ] + [pltpu.VMEM((K, D_FEAT), jnp.float32) for _ in range(NBUF)]
          + [pltpu.VMEM((K,), jnp.int32) for _ in range(3 * NBUF)]
          + [pltpu.VMEM((TAIL,), jnp.int32),
             pltpu.VMEM_SHARED((N_NODES, D_FEAT), jnp.float32)]
          + [pltpu.SemaphoreType.DMA for _ in range(4 * NBUF)],
    )(_sc_body)
    zp = sc_kernel(src, dst, scores[0], scores[1], hd)

    z = pl.pallas_call(
        _add_body,
        out_shape=jax.ShapeDtypeStruct((N_NODES, D_FEAT), jnp.float32),
    )(zp, d[:, None])
    return z
